# baseline (speedup 1.0000x reference)
import functools
import jax
import jax.numpy as jnp
from jax import lax
from jax.experimental import pallas as pl
from jax.experimental.pallas import tpu as pltpu
from jax.experimental.pallas import tpu_sc as plsc

N = 10000
E = 320000
D = 128

NC = 2
NS = 16
NW = NC * NS

CH = 80
NCHUNK = 126
EPT = NCHUNK * CH
EPAD = NW * EPT
RPT = N // NS
ACC_R = N + 8

VCH = 80
VROWS_PT = 640
NV = NW * VROWS_PT

_mesh = plsc.VectorSubcoreMesh(core_axis_name="c", subcore_axis_name="s")


def _segsum_body(h_hbm, src_hbm, dst_hbm, zacc_hbm, zdeg_hbm,
                 acc_out, deg_out,
                 acc_sh, srcb0, srcb1, dstb0, dstb1, rows0, rows1, deg_v,
                 sg0, sg1, sis0, sis1, sid0, sid1):
    cid = lax.axis_index("c")
    sid = lax.axis_index("s")
    pltpu.sync_copy(zacc_hbm, acc_sh.at[pl.ds(sid * RPT, RPT)])
    pltpu.sync_copy(zdeg_hbm, deg_v.at[pl.ds(0, N)])
    ebase = (sid * NC + cid) * EPT
    plsc.subcore_barrier()

    ones = jnp.ones((16,), jnp.float32)
    rows_l = [rows0, rows1]
    sg_l = [sg0, sg1]
    src_l = [srcb0, srcb1]
    dst_l = [dstb0, dstb1]
    sis_l = [sis0, sis1]
    sid_l = [sid0, sid1]

    def istart(i, p):
        off = ebase + i * CH
        pltpu.async_copy(src_hbm.at[pl.ds(off, CH)], src_l[p], sis_l[p])
        pltpu.async_copy(dst_hbm.at[pl.ds(off, CH)], dst_l[p], sid_l[p])

    def iwait(i, p):
        off = ebase + i * CH
        pltpu.make_async_copy(src_hbm.at[pl.ds(off, CH)], src_l[p],
                              sis_l[p]).wait()
        pltpu.make_async_copy(dst_hbm.at[pl.ds(off, CH)], dst_l[p],
                              sid_l[p]).wait()

    def gstart(i, p):
        pltpu.async_copy(h_hbm.at[src_l[p]], rows_l[p], sg_l[p])

    def gwait(i, p):
        pltpu.make_async_copy(h_hbm.at[src_l[p]], rows_l[p], sg_l[p]).wait()

    def deg(p):
        for j in range(CH // 16):
            idx = dst_l[p][pl.ds(j * 16, 16)]
            plsc.addupdate_scatter(deg_v, [idx], ones)

    istart(0, 0)
    istart(1, 1)
    iwait(0, 0)
    gstart(0, 0)
    K2 = NCHUNK // 2

    def step(k, carry):
        i0 = 2 * k
        for c in range(2):
            i = i0 + c
            if c == 0:
                iwait(i + 1, 1)
                gstart(i + 1, 1)
            else:
                @pl.when(k < K2 - 1)
                def _():
                    iwait(i + 1, 0)
                    gstart(i + 1, 0)
            gwait(i, c)
            deg(c)
            pltpu.sync_copy(rows_l[c], acc_sh.at[dst_l[c]], add=True)
            @pl.when(k < K2 - 1)
            def _():
                istart(i + 2, c)
        return carry

    lax.fori_loop(0, K2, step, 0)
    plsc.subcore_barrier()
    pltpu.sync_copy(acc_sh.at[pl.ds(sid * RPT, RPT)],
                    acc_out.at[cid, pl.ds(sid * RPT, RPT)])
    pltpu.sync_copy(deg_v.at[pl.ds(0, N)], deg_out.at[cid, sid])


def _segsum(h_flat, src, dst, zacc, zdeg):
    return pl.kernel(
        _segsum_body,
        out_type=(jax.ShapeDtypeStruct((NC, N, D), jnp.float32),
                  jax.ShapeDtypeStruct((NC, NS, N), jnp.float32)),
        mesh=_mesh,
        scratch_types=[
            pltpu.VMEM_SHARED((ACC_R, D), jnp.float32),
        ] + [pltpu.VMEM((CH,), jnp.int32)] * 4 + [
            pltpu.VMEM((CH, D), jnp.float32),
            pltpu.VMEM((CH, D), jnp.float32),
            pltpu.VMEM((ACC_R,), jnp.float32),
        ] + [pltpu.SemaphoreType.DMA] * 6,
        compiler_params=pltpu.CompilerParams(use_tc_tiling_on_sc=False, needs_layout_passes=False),
    )(h_flat, src, dst, zacc, zdeg)


def _vgather_body(h_hbm, idx_hbm, out_hbm, idx_v, rows0, rows1, sg0, sg1):
    cid = lax.axis_index("c")
    sid = lax.axis_index("s")
    wid = sid * NC + cid
    base = wid * VROWS_PT
    nvc = VROWS_PT // VCH
    rows_l = [rows0, rows1]
    sg_l = [sg0, sg1]
    pltpu.sync_copy(idx_hbm.at[pl.ds(base, VROWS_PT)], idx_v)

    def gstart(c):
        pltpu.async_copy(h_hbm.at[idx_v.at[pl.ds(c * VCH, VCH)]],
                         rows_l[c % 2], sg_l[c % 2])

    def gwait(c):
        pltpu.make_async_copy(h_hbm.at[idx_v.at[pl.ds(c * VCH, VCH)]],
                              rows_l[c % 2], sg_l[c % 2]).wait()

    gstart(0)
    for c in range(nvc):
        if c + 1 < nvc:
            gstart(c + 1)
        gwait(c)
        pltpu.sync_copy(rows_l[c % 2],
                        out_hbm.at[pl.ds(base + c * VCH, VCH)])


def _vgather(h_flat, idx):
    return pl.kernel(
        _vgather_body,
        out_type=jax.ShapeDtypeStruct((NV, D), jnp.float32),
        mesh=_mesh,
        scratch_types=[
            pltpu.VMEM((VROWS_PT,), jnp.int32),
            pltpu.VMEM((VCH, D), jnp.float32),
            pltpu.VMEM((VCH, D), jnp.float32),
            pltpu.SemaphoreType.DMA,
            pltpu.SemaphoreType.DMA,
        ],
        compiler_params=pltpu.CompilerParams(use_tc_tiling_on_sc=False, needs_layout_passes=False),
    )(h_flat, idx)


BN_HOP = 2000


def _hop_body(acc_ref, degp_ref, w_ref, out_ref):
    a = acc_ref[0] + acc_ref[1]
    deg = jnp.sum(degp_ref[...], axis=1)
    h = jnp.dot(a, w_ref[...], preferred_element_type=jnp.float32)
    h = h / jnp.maximum(deg, 1.0)[:, None]
    out_ref[...] = jnp.maximum(h, 0.0)


def _hop_update(acc, degp, w):
    degp_t = jnp.swapaxes(degp.reshape(NW, N), 0, 1)
    nb = N // BN_HOP
    return pl.pallas_call(
        _hop_body,
        grid=(nb,),
        in_specs=[
            pl.BlockSpec((2, BN_HOP, D), lambda b: (0, b, 0)),
            pl.BlockSpec((BN_HOP, NW), lambda b: (b, 0)),
            pl.BlockSpec((D, D), lambda b: (0, 0)),
        ],
        out_specs=pl.BlockSpec((BN_HOP, D), lambda b: (b, 0)),
        out_shape=jax.ShapeDtypeStruct((N, D), jnp.float32),
    )(acc, degp_t, w)


BN_FUS = 2000
NB_FUS = N // BN_FUS


def _fusion_body(v_ref, wi_ref, we_ref, att_ref, fused_ref, loss_ref, s_ref):
    i = pl.program_id(0)

    @pl.when(i == 0)
    def _init():
        for k in range(4):
            s_ref[k] = 0.0

    v0 = v_ref[0]
    v1 = v_ref[1]
    wi = wi_ref[...]
    p0 = jnp.tanh(jnp.dot(v0, wi, preferred_element_type=jnp.float32))
    p1 = jnp.tanh(jnp.dot(v1, wi, preferred_element_type=jnp.float32))
    att = att_ref[0]

    @pl.when(i < NB_FUS)
    def _phase1():
        s_ref[0] += jnp.sum(p0 * att[None, :])
        s_ref[1] += jnp.sum(p1 * att[None, :])
        fused_ref[...] = v0
        loss_ref[0, 0] = 0.0

    @pl.when(i >= NB_FUS)
    def _phase2():
        s0 = s_ref[0] / N
        s1 = s_ref[1] / N
        m = jnp.maximum(s0, s1)
        e0 = jnp.exp(s0 - m)
        e1 = jnp.exp(s1 - m)
        a0 = e0 / (e0 + e1)
        a1 = e1 / (e0 + e1)
        fused = jnp.dot(a0 * v0 + a1 * v1, we_ref[...],
                        preferred_element_type=jnp.float32)
        fused_ref[...] = fused
        re = jnp.sum((fused - p0) ** 2) + jnp.sum((fused - p1) ** 2)
        sq0 = jnp.sum(v0 * v0, axis=1)
        sq1 = jnp.sum(v1 * v1, axis=1)
        inv0 = 1.0 / (jnp.sqrt(sq0) + 1e-8)
        inv1 = 1.0 / (jnp.sqrt(sq1) + 1e-8)
        ortho = jnp.sum(jnp.abs(jnp.sum(v0 * v1, axis=1)) * inv0 * inv1)
        s_ref[2] += re
        s_ref[3] += ortho
        loss_ref[0, 0] = s_ref[2] / (2.0 * N * D) + s_ref[3] / N


def _fusion(views, w_intra, w_inter, att2d):
    return pl.pallas_call(
        _fusion_body,
        grid=(2 * NB_FUS,),
        in_specs=[
            pl.BlockSpec((2, BN_FUS, D), lambda i: (0, lax.rem(i, NB_FUS), 0)),
            pl.BlockSpec((D, D), lambda i: (0, 0)),
            pl.BlockSpec((D, D), lambda i: (0, 0)),
            pl.BlockSpec((1, D), lambda i: (0, 0)),
        ],
        out_specs=[
            pl.BlockSpec((BN_FUS, D), lambda i: (lax.rem(i, NB_FUS), 0)),
            pl.BlockSpec(memory_space=pltpu.SMEM),
        ],
        out_shape=[
            jax.ShapeDtypeStruct((N, D), jnp.float32),
            jax.ShapeDtypeStruct((1, 1), jnp.float32),
        ],
        scratch_shapes=[pltpu.SMEM((4,), jnp.float32)],
    )(views, w_intra, w_inter, att2d)


def kernel(x, W0, W1, W_intra, W_inter, att_v,
           edge_index_mp0_hop0, edge_index_mp0_hop1,
           edge_index_mp1_hop0, edge_index_mp1_hop1,
           feature_index_mp0, feature_index_mp1):
    i32 = jnp.int32
    pad_s = jnp.zeros((EPAD - E,), i32)
    pad_d = jnp.full((EPAD - E,), N, i32)

    def padded(a):
        return jnp.concatenate([a.astype(i32), pad_s])

    def paddedd(a):
        return jnp.concatenate([a.astype(i32), pad_d])

    zacc = jnp.zeros((RPT, D), jnp.float32)
    zdeg = jnp.zeros((N,), jnp.float32)

    acc_a0, deg_a0 = _segsum(x, padded(edge_index_mp0_hop0[0]),
                             paddedd(edge_index_mp0_hop0[1]), zacc, zdeg)
    acc_b0, deg_b0 = _segsum(x, padded(edge_index_mp1_hop0[0]),
                             paddedd(edge_index_mp1_hop0[1]), zacc, zdeg)
    h_a0 = _hop_update(acc_a0, deg_a0, W0)
    h_b0 = _hop_update(acc_b0, deg_b0, W1)
    acc_a1, deg_a1 = _segsum(h_a0, padded(edge_index_mp0_hop1[0]),
                             paddedd(edge_index_mp0_hop1[1]), zacc, zdeg)
    acc_b1, deg_b1 = _segsum(h_b0, padded(edge_index_mp1_hop1[0]),
                             paddedd(edge_index_mp1_hop1[1]), zacc, zdeg)
    h_a1 = _hop_update(acc_a1, deg_a1, W1)
    h_b1 = _hop_update(acc_b1, deg_b1, W0)

    h_all = jnp.concatenate([h_a1, h_b1])
    f_all = jnp.concatenate([
        feature_index_mp0.astype(i32),
        feature_index_mp1.astype(i32) + N,
        jnp.zeros((NV - 2 * N,), i32),
    ])
    v_all = _vgather(h_all, f_all)
    views = v_all[:2 * N].reshape(2, N, D)

    fused, loss = _fusion(views, W_intra, W_inter, att_v.reshape(1, D))
    return fused, loss.reshape(())


# speedup vs baseline: 1.6202x; 1.6202x over previous
import functools
import jax
import jax.numpy as jnp
from jax import lax
from jax.experimental import pallas as pl
from jax.experimental.pallas import tpu as pltpu
from jax.experimental.pallas import tpu_sc as plsc

N = 10000
E = 320000
D = 128

NC = 2
NS = 16
NW = NC * NS

CH = 80
NCHUNK = 250
EPT = NCHUNK * CH
EPAD = NS * EPT
CROWS = EPAD // CH
RPT = N // NS
ACC_R = N + 8

VCH = 80
VROWS_PT = 640
NV = NW * VROWS_PT

_mesh = plsc.VectorSubcoreMesh(core_axis_name="c", subcore_axis_name="s")


def _segsum_body(h_hbm, src_hbm, dst_hbm, zacc_hbm, zdeg_hbm,
                 acc_out, deg_out,
                 acc_sh, srcb0, srcb1, dstb0, dstb1, rows0, rows1, deg_v,
                 sg0, sg1, sis0, sis1, sid0, sid1):
    cid = lax.axis_index("c")
    sid = lax.axis_index("s")
    pltpu.sync_copy(zacc_hbm, acc_sh.at[pl.ds(sid * RPT, RPT)])
    pltpu.sync_copy(zdeg_hbm, deg_v.at[pl.ds(0, N)])
    ebase = cid * EPAD + sid * EPT
    plsc.subcore_barrier()

    ones = jnp.ones((16,), jnp.float32)
    rows_l = [rows0, rows1]
    sg_l = [sg0, sg1]
    src_l = [srcb0, srcb1]
    dst_l = [dstb0, dstb1]
    sis_l = [sis0, sis1]
    sid_l = [sid0, sid1]

    def istart(i, p):
        off = ebase + i * CH
        pltpu.async_copy(src_hbm.at[pl.ds(off, CH)], src_l[p], sis_l[p])
        pltpu.async_copy(dst_hbm.at[pl.ds(off, CH)], dst_l[p], sid_l[p])

    def iwait(i, p):
        off = ebase + i * CH
        pltpu.make_async_copy(src_hbm.at[pl.ds(off, CH)], src_l[p],
                              sis_l[p]).wait()
        pltpu.make_async_copy(dst_hbm.at[pl.ds(off, CH)], dst_l[p],
                              sid_l[p]).wait()

    def gstart(i, p):
        pltpu.async_copy(h_hbm.at[src_l[p]], rows_l[p], sg_l[p])

    def gwait(i, p):
        pltpu.make_async_copy(h_hbm.at[src_l[p]], rows_l[p], sg_l[p]).wait()

    def deg(p):
        for j in range(CH // 16):
            idx = dst_l[p][pl.ds(j * 16, 16)]
            plsc.addupdate_scatter(deg_v, [idx], ones)

    istart(0, 0)
    istart(1, 1)
    iwait(0, 0)
    gstart(0, 0)
    K2 = NCHUNK // 2

    def step(k, carry):
        i0 = 2 * k
        for c in range(2):
            i = i0 + c
            if c == 0:
                iwait(i + 1, 1)
                gstart(i + 1, 1)
            else:
                @pl.when(k < K2 - 1)
                def _():
                    iwait(i + 1, 0)
                    gstart(i + 1, 0)
            gwait(i, c)
            deg(c)
            pltpu.sync_copy(rows_l[c], acc_sh.at[dst_l[c]], add=True)
            @pl.when(k < K2 - 1)
            def _():
                istart(i + 2, c)
        return carry

    lax.fori_loop(0, K2, step, 0)
    plsc.subcore_barrier()
    pltpu.sync_copy(acc_sh.at[pl.ds(sid * RPT, RPT)],
                    acc_out.at[cid, pl.ds(sid * RPT, RPT)])
    pltpu.sync_copy(deg_v.at[pl.ds(0, N)], deg_out.at[cid, sid])


def _segsum(h_flat, src, dst, zacc, zdeg):
    return pl.kernel(
        _segsum_body,
        out_type=(jax.ShapeDtypeStruct((NC, N, D), jnp.float32),
                  jax.ShapeDtypeStruct((NC, NS, N), jnp.float32)),
        mesh=_mesh,
        scratch_types=[
            pltpu.VMEM_SHARED((ACC_R, D), jnp.float32),
        ] + [pltpu.VMEM((CH,), jnp.int32)] * 4 + [
            pltpu.VMEM((CH, D), jnp.float32),
            pltpu.VMEM((CH, D), jnp.float32),
            pltpu.VMEM((ACC_R,), jnp.float32),
        ] + [pltpu.SemaphoreType.DMA] * 6,
        compiler_params=pltpu.CompilerParams(use_tc_tiling_on_sc=False, needs_layout_passes=False),
    )(h_flat, src, dst, zacc, zdeg)


def _vgather_body(h_hbm, idx_hbm, out_hbm, idx_v, rows0, rows1, sg0, sg1):
    cid = lax.axis_index("c")
    sid = lax.axis_index("s")
    wid = sid * NC + cid
    base = wid * VROWS_PT
    nvc = VROWS_PT // VCH
    rows_l = [rows0, rows1]
    sg_l = [sg0, sg1]
    pltpu.sync_copy(idx_hbm.at[pl.ds(base, VROWS_PT)], idx_v)

    def gstart(c):
        pltpu.async_copy(h_hbm.at[idx_v.at[pl.ds(c * VCH, VCH)]],
                         rows_l[c % 2], sg_l[c % 2])

    def gwait(c):
        pltpu.make_async_copy(h_hbm.at[idx_v.at[pl.ds(c * VCH, VCH)]],
                              rows_l[c % 2], sg_l[c % 2]).wait()

    gstart(0)
    for c in range(nvc):
        if c + 1 < nvc:
            gstart(c + 1)
        gwait(c)
        pltpu.sync_copy(rows_l[c % 2],
                        out_hbm.at[pl.ds(base + c * VCH, VCH)])


def _vgather(h_flat, idx):
    return pl.kernel(
        _vgather_body,
        out_type=jax.ShapeDtypeStruct((NV, D), jnp.float32),
        mesh=_mesh,
        scratch_types=[
            pltpu.VMEM((VROWS_PT,), jnp.int32),
            pltpu.VMEM((VCH, D), jnp.float32),
            pltpu.VMEM((VCH, D), jnp.float32),
            pltpu.SemaphoreType.DMA,
            pltpu.SemaphoreType.DMA,
        ],
        compiler_params=pltpu.CompilerParams(use_tc_tiling_on_sc=False, needs_layout_passes=False),
    )(h_flat, idx)


BN_HOP = 2000


def _hop_body(acc_ref, degp_ref, w_ref, out_ref):
    a = acc_ref[0]
    deg = jnp.sum(degp_ref[0], axis=1)
    h = jnp.dot(a, w_ref[0], preferred_element_type=jnp.float32)
    h = h / jnp.maximum(deg, 1.0)[:, None]
    out_ref[0] = jnp.maximum(h, 0.0)


def _hop_update(acc, degp, ws):
    degp_t = jnp.swapaxes(degp, 1, 2)
    nb = N // BN_HOP
    return pl.pallas_call(
        _hop_body,
        grid=(NC, nb),
        in_specs=[
            pl.BlockSpec((1, BN_HOP, D), lambda m, b: (m, b, 0)),
            pl.BlockSpec((1, BN_HOP, NS), lambda m, b: (m, b, 0)),
            pl.BlockSpec((1, D, D), lambda m, b: (m, 0, 0)),
        ],
        out_specs=pl.BlockSpec((1, BN_HOP, D), lambda m, b: (m, b, 0)),
        out_shape=jax.ShapeDtypeStruct((NC, N, D), jnp.float32),
    )(acc, degp_t, ws)


BN_FUS = 2000
NB_FUS = N // BN_FUS


def _fusion_body(v_ref, wi_ref, we_ref, att_ref, fused_ref, loss_ref, s_ref):
    i = pl.program_id(0)

    @pl.when(i == 0)
    def _init():
        for k in range(4):
            s_ref[k] = 0.0

    v0 = v_ref[0]
    v1 = v_ref[1]
    wi = wi_ref[...]
    p0 = jnp.tanh(jnp.dot(v0, wi, preferred_element_type=jnp.float32))
    p1 = jnp.tanh(jnp.dot(v1, wi, preferred_element_type=jnp.float32))
    att = att_ref[0]

    @pl.when(i < NB_FUS)
    def _phase1():
        s_ref[0] += jnp.sum(p0 * att[None, :])
        s_ref[1] += jnp.sum(p1 * att[None, :])
        fused_ref[...] = v0
        loss_ref[0, 0] = 0.0

    @pl.when(i >= NB_FUS)
    def _phase2():
        s0 = s_ref[0] / N
        s1 = s_ref[1] / N
        m = jnp.maximum(s0, s1)
        e0 = jnp.exp(s0 - m)
        e1 = jnp.exp(s1 - m)
        a0 = e0 / (e0 + e1)
        a1 = e1 / (e0 + e1)
        fused = jnp.dot(a0 * v0 + a1 * v1, we_ref[...],
                        preferred_element_type=jnp.float32)
        fused_ref[...] = fused
        re = jnp.sum((fused - p0) ** 2) + jnp.sum((fused - p1) ** 2)
        sq0 = jnp.sum(v0 * v0, axis=1)
        sq1 = jnp.sum(v1 * v1, axis=1)
        inv0 = 1.0 / (jnp.sqrt(sq0) + 1e-8)
        inv1 = 1.0 / (jnp.sqrt(sq1) + 1e-8)
        ortho = jnp.sum(jnp.abs(jnp.sum(v0 * v1, axis=1)) * inv0 * inv1)
        s_ref[2] += re
        s_ref[3] += ortho
        loss_ref[0, 0] = s_ref[2] / (2.0 * N * D) + s_ref[3] / N


def _fusion(views, w_intra, w_inter, att2d):
    return pl.pallas_call(
        _fusion_body,
        grid=(2 * NB_FUS,),
        in_specs=[
            pl.BlockSpec((2, BN_FUS, D), lambda i: (0, lax.rem(i, NB_FUS), 0)),
            pl.BlockSpec((D, D), lambda i: (0, 0)),
            pl.BlockSpec((D, D), lambda i: (0, 0)),
            pl.BlockSpec((1, D), lambda i: (0, 0)),
        ],
        out_specs=[
            pl.BlockSpec((BN_FUS, D), lambda i: (lax.rem(i, NB_FUS), 0)),
            pl.BlockSpec(memory_space=pltpu.SMEM),
        ],
        out_shape=[
            jax.ShapeDtypeStruct((N, D), jnp.float32),
            jax.ShapeDtypeStruct((1, 1), jnp.float32),
        ],
        scratch_shapes=[pltpu.SMEM((4,), jnp.float32)],
    )(views, w_intra, w_inter, att2d)


def kernel(x, W0, W1, W_intra, W_inter, att_v,
           edge_index_mp0_hop0, edge_index_mp0_hop1,
           edge_index_mp1_hop0, edge_index_mp1_hop1,
           feature_index_mp0, feature_index_mp1):
    i32 = jnp.int32
    pad_s = jnp.zeros((EPAD - E,), i32)
    pad_d = jnp.full((EPAD - E,), N, i32)

    def padded(a, b, pad):
        return jnp.concatenate([a, pad, b, pad])

    src_l0 = padded(edge_index_mp0_hop0[0].astype(i32),
                    edge_index_mp1_hop0[0].astype(i32), pad_s)
    dst_l0 = padded(edge_index_mp0_hop0[1].astype(i32),
                    edge_index_mp1_hop0[1].astype(i32), pad_d)
    src_l1 = padded(edge_index_mp0_hop1[0].astype(i32),
                    edge_index_mp1_hop1[0].astype(i32) + N, pad_s)
    dst_l1 = padded(edge_index_mp0_hop1[1].astype(i32),
                    edge_index_mp1_hop1[1].astype(i32), pad_d)
    zacc = jnp.zeros((RPT, D), jnp.float32)
    zdeg = jnp.zeros((N,), jnp.float32)
    ws_l0 = jnp.stack([W0, W1])
    ws_l1 = jnp.stack([W1, W0])

    acc0, degp0 = _segsum(x, src_l0, dst_l0, zacc, zdeg)
    h_l0 = _hop_update(acc0, degp0, ws_l0).reshape(NC * N, D)
    acc1, degp1 = _segsum(h_l0, src_l1, dst_l1, zacc, zdeg)
    h_l1 = _hop_update(acc1, degp1, ws_l1).reshape(NC * N, D)

    f_all = jnp.concatenate([
        feature_index_mp0.astype(i32),
        feature_index_mp1.astype(i32) + N,
        jnp.zeros((NV - 2 * N,), i32),
    ])
    v_all = _vgather(h_l1, f_all)
    views = v_all[:2 * N].reshape(2, N, D)

    fused, loss = _fusion(views, W_intra, W_inter, att_v.reshape(1, D))
    return fused, loss.reshape(())
